# Initial kernel scaffold; baseline (speedup 1.0000x reference)
import jax
import jax.numpy as jnp
from jax.experimental import pallas as pl


def kernel(scores, encodings):
    raise NotImplementedError("write your pallas kernel here")


# speedup vs baseline: 2.0434x; 2.0434x over previous
import functools

import jax
import jax.numpy as jnp
from jax import lax
from jax.experimental import pallas as pl
from jax.experimental.pallas import tpu as pltpu
from jax.experimental.pallas import tpu_sc as plsc

BATCH = 8
NUM_BOXES = 20000
UNITS = 128

_INFO = plsc.get_sparse_core_info()
_NC, _NS, _L = _INFO.num_cores, _INFO.num_subcores, _INFO.num_lanes
_NW = _NC * _NS
_ROWS = BATCH * NUM_BOXES
_R_PER_W = _ROWS // _NW
_W = 200
_NWIN = _R_PER_W // _W


def _gather_kernel(enc_hbm, idx_hbm, out_hbm, idx_v, rows_v, sem):
    wid = lax.axis_index("s") * _NC + lax.axis_index("c")
    base = wid * _R_PER_W
    pltpu.sync_copy(idx_hbm.at[pl.ds(base, _R_PER_W)], idx_v)

    def body(w, carry):
        off = w * _W
        pltpu.async_copy(enc_hbm.at[idx_v.at[pl.ds(off, _W)]], rows_v, sem).wait()
        pltpu.sync_copy(rows_v, out_hbm.at[pl.ds(base + off, _W)])
        return carry

    lax.fori_loop(0, _NWIN, body, 0)


@jax.jit
def kernel(scores, encodings):
    perm = jnp.argsort(-scores, axis=-1).astype(jnp.int32)
    idx_flat = perm.reshape(-1)
    mesh = plsc.VectorSubcoreMesh(core_axis_name="c", subcore_axis_name="s")
    out = pl.kernel(
        _gather_kernel,
        mesh=mesh,
        out_type=jax.ShapeDtypeStruct((_ROWS, UNITS), jnp.float32),
        scratch_types=[
            pltpu.VMEM((_R_PER_W,), jnp.int32),
            pltpu.VMEM((_W, UNITS), jnp.float32),
            pltpu.SemaphoreType.DMA,
        ],
    )(encodings, idx_flat)
    return lax.stop_gradient(out.reshape(BATCH, NUM_BOXES, UNITS))


# speedup vs baseline: 2.3689x; 1.1593x over previous
import jax
import jax.numpy as jnp
from jax import lax
from jax.experimental import pallas as pl
from jax.experimental.pallas import tpu as pltpu
from jax.experimental.pallas import tpu_sc as plsc

BATCH = 8
NUM_BOXES = 20000
UNITS = 128

_INFO = plsc.get_sparse_core_info()
_NC, _NS, _L = _INFO.num_cores, _INFO.num_subcores, _INFO.num_lanes
_ROWS = BATCH * NUM_BOXES
_ROWS_PER_CORE = BATCH // _NC
_BLK = NUM_BOXES // _L
_NBINS = 256
_HIST = _NBINS * _L
_G = NUM_BOXES // _NS
_GPT = (_ROWS_PER_CORE * NUM_BOXES) // _NS
_W = 200
_NWIN = _GPT // _W
_PARTS = NUM_BOXES // _GPT

_INT_MIN = -2147483648
_POS_XOR = 0x7FFFFFFF


def _desc_key(raw_bits):
    u = jnp.where(raw_bits == _INT_MIN, 0, raw_bits)
    return jnp.where(u < 0, u, u ^ _POS_XOR)


def _sc_kernel(scores_hbm, enc_hbm, out_hbm,
               keys_a, keys_b, vals_a, vals_b, hist,
               idx_v, rows_v, spmem, sem):
    c = lax.axis_index("c")
    s = lax.axis_index("s")
    lane = jnp.arange(_L, dtype=jnp.int32)
    lane_off = lane * _BLK
    ones = jnp.ones((_L,), jnp.int32)
    zeros = jnp.zeros((_L,), jnp.int32)

    @pl.when(s < _ROWS_PER_CORE)
    def _sort():
        row = c * _ROWS_PER_CORE + s
        pltpu.sync_copy(scores_hbm.at[row], keys_a)

        passes = [
            (keys_a, None, keys_b, vals_b),
            (keys_b, vals_b, keys_a, vals_a),
            (keys_a, vals_a, keys_b, vals_b),
            (keys_b, vals_b, None, vals_a),
        ]
        for p, (src_k, src_v, dst_k, dst_v) in enumerate(passes):
            shift = jnp.int32(8 * p)

            def _key_at(t):
                k = plsc.load_gather(src_k, [lane_off + t])
                if p == 0:
                    k = _desc_key(k)
                return k

            def _zero(i, _):
                hist[pl.ds(i * _L, _L)] = zeros
                return 0
            lax.fori_loop(0, _NBINS, _zero, 0)

            def _hist(t, _):
                d = lax.shift_right_logical(_key_at(t), shift) & (_NBINS - 1)
                plsc.addupdate_scatter(hist, [d * _L + lane], ones)
                return 0
            lax.fori_loop(0, _BLK, _hist, 0)

            def _scan(i, carry):
                v = hist[pl.ds(i * _L, _L)]
                incl = plsc.cumsum(v)
                hist[pl.ds(i * _L, _L)] = incl - v + carry
                return carry + jnp.sum(v)
            lax.fori_loop(0, _NBINS, _scan, jnp.int32(0))

            def _place(t, _):
                k = _key_at(t)
                lidx = lane_off + t
                v = lidx if src_v is None else plsc.load_gather(src_v, [lidx])
                d = lax.shift_right_logical(k, shift) & (_NBINS - 1)
                oidx = d * _L + lane
                off = plsc.load_gather(hist, [oidx])
                if dst_k is not None:
                    plsc.store_scatter(dst_k, [off], k)
                plsc.store_scatter(dst_v, [off], v)
                plsc.addupdate_scatter(hist, [oidx], ones)
                return 0
            lax.fori_loop(0, _BLK, _place, 0)

        pltpu.sync_copy(vals_a, spmem.at[pl.ds(s * NUM_BOXES, NUM_BOXES)])

    plsc.subcore_barrier()

    row_local = s // _PARTS
    part = s % _PARTS
    pltpu.sync_copy(spmem.at[pl.ds(row_local * NUM_BOXES + part * _GPT, _GPT)], idx_v)
    out_base = (c * _ROWS_PER_CORE + row_local) * NUM_BOXES + part * _GPT

    def _gather(w, _):
        off = w * _W
        pltpu.async_copy(enc_hbm.at[idx_v.at[pl.ds(off, _W)]], rows_v, sem).wait()
        pltpu.sync_copy(rows_v, out_hbm.at[pl.ds(out_base + off, _W)])
        return 0
    lax.fori_loop(0, _NWIN, _gather, 0)


@jax.jit
def kernel(scores, encodings):
    scores_bits = lax.bitcast_convert_type(scores, jnp.int32)
    mesh = plsc.VectorSubcoreMesh(core_axis_name="c", subcore_axis_name="s")
    out = pl.kernel(
        _sc_kernel,
        mesh=mesh,
        compiler_params=pltpu.CompilerParams(needs_layout_passes=False),
        out_type=jax.ShapeDtypeStruct((_ROWS, UNITS), jnp.float32),
        scratch_types=[
            pltpu.VMEM((NUM_BOXES,), jnp.int32),
            pltpu.VMEM((NUM_BOXES,), jnp.int32),
            pltpu.VMEM((NUM_BOXES,), jnp.int32),
            pltpu.VMEM((NUM_BOXES,), jnp.int32),
            pltpu.VMEM((_HIST,), jnp.int32),
            pltpu.VMEM((_GPT,), jnp.int32),
            pltpu.VMEM((_W, UNITS), jnp.float32),
            pltpu.VMEM_SHARED((_ROWS_PER_CORE * NUM_BOXES,), jnp.int32),
            pltpu.SemaphoreType.DMA,
        ],
    )(scores_bits, encodings)
    return lax.stop_gradient(out.reshape(BATCH, NUM_BOXES, UNITS))


# speedup vs baseline: 2.5707x; 1.0852x over previous
import jax
import jax.numpy as jnp
from jax import lax
from jax.experimental import pallas as pl
from jax.experimental.pallas import tpu as pltpu
from jax.experimental.pallas import tpu_sc as plsc

BATCH = 8
NUM_BOXES = 20000
UNITS = 128

_INFO = plsc.get_sparse_core_info()
_NC, _NS, _L = _INFO.num_cores, _INFO.num_subcores, _INFO.num_lanes
_ROWS = BATCH * NUM_BOXES
_ROWS_PER_CORE = BATCH // _NC
_BLK = NUM_BOXES // _L
_NBINS = 256
_HIST = _NBINS * _L
_G = NUM_BOXES // _NS
_GPT = (_ROWS_PER_CORE * NUM_BOXES) // _NS
_W = 40
_NBUF = 5
_NWIN = _GPT // _W
_PARTS = NUM_BOXES // _GPT

_INT_MIN = -2147483648
_POS_XOR = 0x7FFFFFFF


def _desc_key(raw_bits):
    u = jnp.where(raw_bits == _INT_MIN, 0, raw_bits)
    return jnp.where(u < 0, u, u ^ _POS_XOR)


def _sc_kernel(scores_hbm, enc_hbm, out_hbm,
               keys_a, keys_b, vals_a, vals_b, hist,
               idx_v, rows_v, spmem, gsems, ssems):
    c = lax.axis_index("c")
    s = lax.axis_index("s")
    lane = jnp.arange(_L, dtype=jnp.int32)
    lane_off = lane * _BLK
    ones = jnp.ones((_L,), jnp.int32)
    zeros = jnp.zeros((_L,), jnp.int32)

    @pl.when(s < _ROWS_PER_CORE)
    def _sort():
        row = c * _ROWS_PER_CORE + s
        pltpu.sync_copy(scores_hbm.at[row], keys_a)

        passes = [
            (keys_a, None, keys_b, vals_b),
            (keys_b, vals_b, keys_a, vals_a),
            (keys_a, vals_a, keys_b, vals_b),
            (keys_b, vals_b, None, vals_a),
        ]
        for p, (src_k, src_v, dst_k, dst_v) in enumerate(passes):
            shift = jnp.int32(8 * p)

            def _key_at(t):
                k = plsc.load_gather(src_k, [lane_off + t])
                if p == 0:
                    k = _desc_key(k)
                return k

            def _zero(i, _):
                hist[pl.ds(i * _L, _L)] = zeros
                return 0
            lax.fori_loop(0, _NBINS, _zero, 0)

            def _hist(t, _):
                d = lax.shift_right_logical(_key_at(t), shift) & (_NBINS - 1)
                plsc.addupdate_scatter(hist, [d * _L + lane], ones)
                return 0
            lax.fori_loop(0, _BLK, _hist, 0)

            def _scan(i, carry):
                v = hist[pl.ds(i * _L, _L)]
                incl = plsc.cumsum(v)
                hist[pl.ds(i * _L, _L)] = incl - v + carry
                return carry + jnp.sum(v)
            lax.fori_loop(0, _NBINS, _scan, jnp.int32(0))

            def _place(t, _):
                k = _key_at(t)
                lidx = lane_off + t
                v = lidx if src_v is None else plsc.load_gather(src_v, [lidx])
                d = lax.shift_right_logical(k, shift) & (_NBINS - 1)
                oidx = d * _L + lane
                off = plsc.load_gather(hist, [oidx])
                if dst_k is not None:
                    plsc.store_scatter(dst_k, [off], k)
                plsc.store_scatter(dst_v, [off], v)
                plsc.addupdate_scatter(hist, [oidx], ones)
                return 0
            lax.fori_loop(0, _BLK, _place, 0)

        pltpu.sync_copy(vals_a, spmem.at[pl.ds(s * NUM_BOXES, NUM_BOXES)])

    plsc.subcore_barrier()

    row_local = s // _PARTS
    part = s % _PARTS
    pltpu.sync_copy(spmem.at[pl.ds(row_local * NUM_BOXES + part * _GPT, _GPT)], idx_v)
    out_base = (c * _ROWS_PER_CORE + row_local) * NUM_BOXES + part * _GPT

    def _start_g(w, b):
        pltpu.async_copy(
            enc_hbm.at[idx_v.at[pl.ds(w * _W, _W)]], rows_v[b], gsems[b])

    for b in range(_NBUF):
        _start_g(b, b)

    def _outer(g, _):
        for b in range(_NBUF):
            w = g * _NBUF + b
            pltpu.make_async_copy(
                enc_hbm.at[idx_v.at[pl.ds(w * _W, _W)]], rows_v[b], gsems[b]
            ).wait()
            copy = pltpu.async_copy(
                rows_v[b], out_hbm.at[pl.ds(out_base + w * _W, _W)], ssems[b])
            copy.wait()

            @pl.when(w + _NBUF < _NWIN)
            def _():
                _start_g(w + _NBUF, b)
        return 0
    lax.fori_loop(0, _NWIN // _NBUF, _outer, 0)


@jax.jit
def kernel(scores, encodings):
    scores_bits = lax.bitcast_convert_type(scores, jnp.int32)
    mesh = plsc.VectorSubcoreMesh(core_axis_name="c", subcore_axis_name="s")
    out = pl.kernel(
        _sc_kernel,
        mesh=mesh,
        compiler_params=pltpu.CompilerParams(needs_layout_passes=False),
        out_type=jax.ShapeDtypeStruct((_ROWS, UNITS), jnp.float32),
        scratch_types=[
            pltpu.VMEM((NUM_BOXES,), jnp.int32),
            pltpu.VMEM((NUM_BOXES,), jnp.int32),
            pltpu.VMEM((NUM_BOXES,), jnp.int32),
            pltpu.VMEM((NUM_BOXES,), jnp.int32),
            pltpu.VMEM((_HIST,), jnp.int32),
            pltpu.VMEM((_GPT,), jnp.int32),
            [pltpu.VMEM((_W, UNITS), jnp.float32) for _ in range(_NBUF)],
            pltpu.VMEM_SHARED((_ROWS_PER_CORE * NUM_BOXES,), jnp.int32),
            [pltpu.SemaphoreType.DMA for _ in range(_NBUF)],
            [pltpu.SemaphoreType.DMA for _ in range(_NBUF)],
        ],
    )(scores_bits, encodings)
    return lax.stop_gradient(out.reshape(BATCH, NUM_BOXES, UNITS))


# speedup vs baseline: 3.3794x; 1.3146x over previous
import struct

import jax
import jax.numpy as jnp
from jax import lax
from jax.experimental import pallas as pl
from jax.experimental.pallas import tpu as pltpu
from jax.experimental.pallas import tpu_sc as plsc

BATCH = 8
NUM_BOXES = 20000
UNITS = 128

_INFO = plsc.get_sparse_core_info()
_NC, _NS, _L = _INFO.num_cores, _INFO.num_subcores, _INFO.num_lanes
_ROWS = BATCH * NUM_BOXES
_ROWS_PER_CORE = BATCH // _NC
_PARTS = _NS // _ROWS_PER_CORE
_VBLK = NUM_BOXES // _L
_NBINS = 256
_HIST = _NBINS * _L
_CAP = NUM_BOXES + _L
_W = 48
_NBUF = 5

_INT_MIN = -2147483648


def _desc_key(raw_bits):
    u = jnp.where(raw_bits == _INT_MIN, 0, raw_bits)
    return jnp.where(u < 0, u, u ^ 0x7FFFFFFF)


def _skey_of(score):
    u = struct.unpack("<i", struct.pack("<f", score))[0]
    k = u if u < 0 else u ^ 0x7FFFFFFF
    s = (k ^ 0x80000000) & 0xFFFFFFFF
    return s - (1 << 32) if s >= (1 << 31) else s


_PIVOTS = [0.6744897501960817, 0.0, -0.6744897501960817]
_BOUNDS = [_skey_of(p) for p in _PIVOTS]


def _sc_kernel(scores_hbm, enc_hbm, out_hbm,
               keys_a, keys_b, vals_a, vals_b, hist,
               cnt_stage, cnt_all, rows_v, row1, oidx_v,
               counts_sp, gsems, ssems, tsem):
    c = lax.axis_index("c")
    s = lax.axis_index("s")
    row_local = s // _PARTS
    q = s % _PARTS
    row = c * _ROWS_PER_CORE + row_local
    lane = jnp.arange(_L, dtype=jnp.int32)
    ones = jnp.ones((_L,), jnp.int32)
    zeros = jnp.zeros((_L,), jnp.int32)

    pltpu.sync_copy(scores_hbm.at[row], keys_b)

    lo = jnp.where(q == 0, _INT_MIN, 0)
    for i, b in enumerate(_BOUNDS):
        lo = jnp.where(q == i + 1, b, lo)
    hi = jnp.where(q == _PARTS - 1, 0x7FFFFFFF, 0)
    for i, b in enumerate(_BOUNDS):
        hi = jnp.where(q == i, b, hi)

    def _filter(t, n):
        k = _desc_key(keys_b[pl.ds(t * _L, _L)])
        sk = k ^ _INT_MIN
        m = (sk >= lo) & (sk < hi)
        plsc.store_compressed(keys_a.at[pl.ds(n, _L)], k, mask=m)
        plsc.store_compressed(vals_a.at[pl.ds(n, _L)], t * _L + lane, mask=m)
        return n + jnp.sum(jnp.where(m, 1, 0))
    n = lax.fori_loop(0, _VBLK, _filter, jnp.int32(0))

    keys_a[pl.ds(n, _L)] = jnp.full((_L,), -1, jnp.int32)
    vals_a[pl.ds(n, _L)] = zeros
    nblk = (n + _L - 1) // _L

    cnt_stage[pl.ds(0, _L)] = jnp.full((_L,), n, jnp.int32)
    pltpu.sync_copy(cnt_stage.at[pl.ds(0, 8)], counts_sp.at[pl.ds(s * 8, 8)])
    plsc.subcore_barrier()
    pltpu.sync_copy(counts_sp, cnt_all)
    counts16 = plsc.load_gather(cnt_all, [lane * 8])
    sel = (lane >= row_local * _PARTS) & (lane < row_local * _PARTS + q)
    base = jnp.sum(jnp.where(sel, counts16, 0))

    passes = [
        (keys_a, vals_a, keys_b, vals_b),
        (keys_b, vals_b, keys_a, vals_a),
        (keys_a, vals_a, keys_b, vals_b),
        (keys_b, vals_b, None, vals_a),
    ]
    for p, (src_k, src_v, dst_k, dst_v) in enumerate(passes):
        shift = jnp.int32(8 * p)

        def _zero(i, _):
            hist[pl.ds(i * _L, _L)] = zeros
            return 0
        lax.fori_loop(0, _NBINS, _zero, 0)

        def _hist(t, _):
            k = plsc.load_gather(src_k, [lane * nblk + t])
            d = lax.shift_right_logical(k, shift) & (_NBINS - 1)
            plsc.addupdate_scatter(hist, [d * _L + lane], ones)
            return 0
        lax.fori_loop(0, nblk, _hist, 0)

        def _scan(i, carry):
            v = hist[pl.ds(i * _L, _L)]
            incl = plsc.cumsum(v)
            hist[pl.ds(i * _L, _L)] = incl - v + carry
            return carry + jnp.sum(v)
        lax.fori_loop(0, _NBINS, _scan, jnp.int32(0))

        def _place(t, _):
            lidx = lane * nblk + t
            k = plsc.load_gather(src_k, [lidx])
            v = plsc.load_gather(src_v, [lidx])
            d = lax.shift_right_logical(k, shift) & (_NBINS - 1)
            oidx = d * _L + lane
            off = plsc.load_gather(hist, [oidx])
            if dst_k is not None:
                plsc.store_scatter(dst_k, [off], k)
            plsc.store_scatter(dst_v, [off], v)
            plsc.addupdate_scatter(hist, [oidx], ones)
            return 0
        lax.fori_loop(0, nblk, _place, 0)

    out_base = row * NUM_BOXES + base
    nwin = n // _W

    def _start_g(w, b):
        pltpu.async_copy(
            enc_hbm.at[vals_a.at[pl.ds(w * _W, _W)]], rows_v[b], gsems[b])

    for b in range(_NBUF):
        @pl.when(b < nwin)
        def _():
            _start_g(b, b)

    def _outer(g, _):
        for b in range(_NBUF):
            w = g * _NBUF + b

            @pl.when(w < nwin)
            def _():
                pltpu.make_async_copy(
                    enc_hbm.at[vals_a.at[pl.ds(w * _W, _W)]], rows_v[b],
                    gsems[b]).wait()
                for i in range(_W // _L):
                    oidx_v[pl.ds(i * _L, _L)] = out_base + w * _W + i * _L + lane
                pltpu.async_copy(
                    rows_v[b], out_hbm.at[oidx_v], ssems[b]).wait()

                @pl.when(w + _NBUF < nwin)
                def _():
                    _start_g(w + _NBUF, b)
        return 0
    lax.fori_loop(0, (nwin + _NBUF - 1) // _NBUF, _outer, 0)

    def _tail(t, _):
        gidx = plsc.load_gather(vals_a, [jnp.full((_L,), t, jnp.int32)])
        oidx_v[pl.ds(0, _L)] = gidx
        oidx_v[pl.ds(_L, _L)] = jnp.full((_L,), out_base + t, jnp.int32)
        pltpu.async_copy(
            enc_hbm.at[oidx_v.at[pl.ds(0, 1)]], row1, tsem).wait()
        pltpu.async_copy(
            row1, out_hbm.at[oidx_v.at[pl.ds(_L, 1)]], tsem).wait()
        return 0
    lax.fori_loop(nwin * _W, n, _tail, 0)


@jax.jit
def kernel(scores, encodings):
    scores_bits = lax.bitcast_convert_type(scores, jnp.int32)
    mesh = plsc.VectorSubcoreMesh(core_axis_name="c", subcore_axis_name="s")
    out = pl.kernel(
        _sc_kernel,
        mesh=mesh,
        compiler_params=pltpu.CompilerParams(needs_layout_passes=False),
        out_type=jax.ShapeDtypeStruct((_ROWS, UNITS), jnp.float32),
        scratch_types=[
            pltpu.VMEM((_CAP,), jnp.int32),
            pltpu.VMEM((NUM_BOXES,), jnp.int32),
            pltpu.VMEM((_CAP,), jnp.int32),
            pltpu.VMEM((NUM_BOXES,), jnp.int32),
            pltpu.VMEM((_HIST,), jnp.int32),
            pltpu.VMEM((_L,), jnp.int32),
            pltpu.VMEM((8 * _NS,), jnp.int32),
            [pltpu.VMEM((_W, UNITS), jnp.float32) for _ in range(_NBUF)],
            pltpu.VMEM((1, UNITS), jnp.float32),
            pltpu.VMEM((_W,), jnp.int32),
            pltpu.VMEM_SHARED((8 * _NS,), jnp.int32),
            [pltpu.SemaphoreType.DMA for _ in range(_NBUF)],
            [pltpu.SemaphoreType.DMA for _ in range(_NBUF)],
            pltpu.SemaphoreType.DMA,
        ],
    )(scores_bits, encodings)
    return lax.stop_gradient(out.reshape(BATCH, NUM_BOXES, UNITS))
